# baseline (speedup 1.0000x reference)
import functools

import jax
import jax.numpy as jnp
from jax import lax
from jax.experimental import pallas as pl
from jax.experimental.pallas import tpu as pltpu
from jax.experimental.pallas import tpu_sc as plsc

N_NODES = 10000
D = 128
DH = D // 2

NC = 2
NS = 16
CHUNK = 128
NBUF = 2
GRP = 2

N_PAD = 10112
ZROWS = N_PAD // NS


def _make_agg(n_chunks):
    mesh = plsc.VectorSubcoreMesh(core_axis_name="c", subcore_axis_name="s")

    @functools.partial(
        pl.kernel,
        out_type=jax.ShapeDtypeStruct((NC, N_PAD, DH), jnp.float32),
        mesh=mesh,
        compiler_params=pltpu.CompilerParams(use_tc_tiling_on_sc=False),
        scratch_types=[
            pltpu.VMEM((n_chunks // GRP, GRP * CHUNK), jnp.int32),
            pltpu.VMEM((n_chunks // GRP, GRP * CHUNK), jnp.int32),
            [pltpu.VMEM((GRP * CHUNK, DH), jnp.float32) for _ in range(NBUF)],
            pltpu.VMEM_SHARED((N_PAD, DH), jnp.float32),
            [pltpu.SemaphoreType.DMA for _ in range(NBUF)],
            [pltpu.SemaphoreType.DMA for _ in range(NBUF)],
        ],
    )
    def agg(src_hbm, dst_hbm, feat_hbm, zeros_hbm, out_hbm,
            src_v, dst_v, rows, acc, gsem, ssem):
        cid = lax.axis_index("c")
        sid = lax.axis_index("s")

        pltpu.sync_copy(zeros_hbm, acc.at[pl.ds(sid * ZROWS, ZROWS)])
        pltpu.sync_copy(src_hbm.at[cid, sid], src_v)
        pltpu.sync_copy(dst_hbm.at[sid], dst_v)
        plsc.subcore_barrier()

        def body(i, _):
            base = NBUF * i
            gs = [
                pltpu.async_copy(
                    feat_hbm.at[src_v.at[base + k]], rows[k], gsem[k])
                for k in range(NBUF)
            ]
            ss = []
            for k in range(NBUF):
                gs[k].wait()
                ss.append(pltpu.async_copy(
                    rows[k], acc.at[dst_v.at[base + k]], ssem[k],
                    add=True))
            for k in range(NBUF):
                ss[k].wait()
            return 0

        lax.fori_loop(0, n_chunks // (NBUF * GRP), body, 0)
        plsc.subcore_barrier()

        pltpu.sync_copy(
            acc.at[pl.ds(sid * ZROWS, ZROWS)],
            out_hbm.at[cid, pl.ds(sid * ZROWS, ZROWS)],
        )

    return agg


def _dense_body(acc_ref, feat_ref, wn_ref, ws_ref, b_ref, out_ref):
    out_ref[...] = (
        jnp.dot(acc_ref[0], wn_ref[0:DH, :], preferred_element_type=jnp.float32)
        + jnp.dot(acc_ref[1], wn_ref[DH:D, :], preferred_element_type=jnp.float32)
        + jnp.dot(feat_ref[...], ws_ref[...], preferred_element_type=jnp.float32)
        + b_ref[...]
    )


def _make_dense(blk, n_blk):
    return pl.pallas_call(
        _dense_body,
        grid=(n_blk,),
        in_specs=[
            pl.BlockSpec((NC, blk, DH), lambda i: (0, i, 0)),
            pl.BlockSpec((blk, D), lambda i: (i, 0)),
            pl.BlockSpec((D, D), lambda i: (0, 0)),
            pl.BlockSpec((D, D), lambda i: (0, 0)),
            pl.BlockSpec((1, D), lambda i: (0, 0)),
        ],
        out_specs=pl.BlockSpec((blk, D), lambda i: (i, 0)),
        out_shape=jax.ShapeDtypeStruct((N_NODES, D), jnp.float32),
    )


def kernel(feat, edge_index, W_neigh, b_neigh, W_self):
    src = edge_index[0].astype(jnp.int32)
    dst = edge_index[1].astype(jnp.int32)
    n_edges = src.shape[0]

    per_tile = -(-n_edges // NS)
    n_chunks = -(-per_tile // (CHUNK * NBUF * GRP)) * (NBUF * GRP)
    e_pad = NS * n_chunks * CHUNK

    pad = e_pad - n_edges
    pad_idx = jnp.arange(pad, dtype=jnp.int32)
    src_p = jnp.concatenate([src, pad_idx % N_NODES])
    src_all = jnp.stack([2 * src_p, 2 * src_p + 1]).reshape(
        NC, NS, n_chunks // GRP, GRP * CHUNK)
    dst_p = jnp.concatenate(
        [dst, N_NODES + (pad_idx & 63)]
    ).reshape(NS, n_chunks // GRP, GRP * CHUNK)

    feat_half = feat.reshape(NC * N_NODES, DH)
    zeros = jnp.zeros((ZROWS, DH), jnp.float32)
    acc = _make_agg(n_chunks)(src_all, dst_p, feat_half, zeros)

    blk = 1000
    n_blk = N_NODES // blk
    return _make_dense(blk, n_blk)(acc, feat, W_neigh, W_self,
                                   b_neigh.reshape(1, D))


# speedup vs baseline: 1.0044x; 1.0044x over previous
import functools

import jax
import jax.numpy as jnp
from jax import lax
from jax.experimental import pallas as pl
from jax.experimental.pallas import tpu as pltpu
from jax.experimental.pallas import tpu_sc as plsc

N_NODES = 10000
D = 128
DH = D // 2

NC = 2
NS = 16
CHUNK = 128
NBUF = 3
GRP = 2

N_PAD = 10112
ZROWS = N_PAD // NS


def _make_agg(n_chunks):
    mesh = plsc.VectorSubcoreMesh(core_axis_name="c", subcore_axis_name="s")

    @functools.partial(
        pl.kernel,
        out_type=jax.ShapeDtypeStruct((NC, N_PAD, DH), jnp.float32),
        mesh=mesh,
        compiler_params=pltpu.CompilerParams(use_tc_tiling_on_sc=False),
        scratch_types=[
            pltpu.VMEM((n_chunks // GRP, GRP * CHUNK), jnp.int32),
            pltpu.VMEM((n_chunks // GRP, GRP * CHUNK), jnp.int32),
            [pltpu.VMEM((GRP * CHUNK, DH), jnp.float32) for _ in range(NBUF)],
            pltpu.VMEM_SHARED((N_PAD, DH), jnp.float32),
            [pltpu.SemaphoreType.DMA for _ in range(NBUF)],
            [pltpu.SemaphoreType.DMA for _ in range(NBUF)],
        ],
    )
    def agg(src_hbm, dst_hbm, feat_hbm, zeros_hbm, out_hbm,
            src_v, dst_v, rows, acc, gsem, ssem):
        cid = lax.axis_index("c")
        sid = lax.axis_index("s")

        pltpu.sync_copy(zeros_hbm, acc.at[pl.ds(sid * ZROWS, ZROWS)])
        pltpu.sync_copy(src_hbm.at[cid, sid], src_v)
        pltpu.sync_copy(dst_hbm.at[sid], dst_v)
        plsc.subcore_barrier()

        def body(i, _):
            base = NBUF * i
            gs = [
                pltpu.async_copy(
                    feat_hbm.at[src_v.at[base + k]], rows[k], gsem[k])
                for k in range(NBUF)
            ]
            ss = []
            for k in range(NBUF):
                gs[k].wait()
                ss.append(pltpu.async_copy(
                    rows[k], acc.at[dst_v.at[base + k]], ssem[k],
                    add=True))
            for k in range(NBUF):
                ss[k].wait()
            return 0

        lax.fori_loop(0, n_chunks // (NBUF * GRP), body, 0)
        plsc.subcore_barrier()

        pltpu.sync_copy(
            acc.at[pl.ds(sid * ZROWS, ZROWS)],
            out_hbm.at[cid, pl.ds(sid * ZROWS, ZROWS)],
        )

    return agg


def _dense_body(acc_ref, feat_ref, wn_ref, ws_ref, b_ref, out_ref):
    out_ref[...] = (
        jnp.dot(acc_ref[0], wn_ref[0:DH, :], preferred_element_type=jnp.float32)
        + jnp.dot(acc_ref[1], wn_ref[DH:D, :], preferred_element_type=jnp.float32)
        + jnp.dot(feat_ref[...], ws_ref[...], preferred_element_type=jnp.float32)
        + b_ref[...]
    )


def _make_dense(blk, n_blk):
    return pl.pallas_call(
        _dense_body,
        grid=(n_blk,),
        in_specs=[
            pl.BlockSpec((NC, blk, DH), lambda i: (0, i, 0)),
            pl.BlockSpec((blk, D), lambda i: (i, 0)),
            pl.BlockSpec((D, D), lambda i: (0, 0)),
            pl.BlockSpec((D, D), lambda i: (0, 0)),
            pl.BlockSpec((1, D), lambda i: (0, 0)),
        ],
        out_specs=pl.BlockSpec((blk, D), lambda i: (i, 0)),
        out_shape=jax.ShapeDtypeStruct((N_NODES, D), jnp.float32),
    )


def kernel(feat, edge_index, W_neigh, b_neigh, W_self):
    src = edge_index[0].astype(jnp.int32)
    dst = edge_index[1].astype(jnp.int32)
    n_edges = src.shape[0]

    per_tile = -(-n_edges // NS)
    n_chunks = -(-per_tile // (CHUNK * NBUF * GRP)) * (NBUF * GRP)
    e_pad = NS * n_chunks * CHUNK

    pad = e_pad - n_edges
    pad_idx = jnp.arange(pad, dtype=jnp.int32)
    src_p = jnp.concatenate([src, pad_idx % N_NODES])
    src_all = jnp.stack([2 * src_p, 2 * src_p + 1]).reshape(
        NC, NS, n_chunks // GRP, GRP * CHUNK)
    dst_p = jnp.concatenate(
        [dst, N_NODES + (pad_idx & 63)]
    ).reshape(NS, n_chunks // GRP, GRP * CHUNK)

    feat_half = feat.reshape(NC * N_NODES, DH)
    zeros = jnp.zeros((ZROWS, DH), jnp.float32)
    acc = _make_agg(n_chunks)(src_all, dst_p, feat_half, zeros)

    blk = 1000
    n_blk = N_NODES // blk
    return _make_dense(blk, n_blk)(acc, feat, W_neigh, W_self,
                                   b_neigh.reshape(1, D))


# speedup vs baseline: 1.2125x; 1.2071x over previous
import functools

import jax
import jax.numpy as jnp
from jax import lax
from jax.experimental import pallas as pl
from jax.experimental.pallas import tpu as pltpu
from jax.experimental.pallas import tpu_sc as plsc

N_NODES = 10000
D = 128
DH = D // 2

NC = 2
NS = 16
CHUNK = 128
NBUF = 6
GRP = 1

N_PAD = 10112
ZROWS = N_PAD // NS


def _make_agg(n_chunks):
    mesh = plsc.VectorSubcoreMesh(core_axis_name="c", subcore_axis_name="s")

    @functools.partial(
        pl.kernel,
        out_type=jax.ShapeDtypeStruct((NC, N_PAD, DH), jnp.float32),
        mesh=mesh,
        compiler_params=pltpu.CompilerParams(use_tc_tiling_on_sc=False),
        scratch_types=[
            pltpu.VMEM((n_chunks // GRP, GRP * CHUNK), jnp.int32),
            pltpu.VMEM((n_chunks // GRP, GRP * CHUNK), jnp.int32),
            [pltpu.VMEM((GRP * CHUNK, DH), jnp.float32) for _ in range(NBUF)],
            pltpu.VMEM_SHARED((N_PAD, DH), jnp.float32),
            [pltpu.SemaphoreType.DMA for _ in range(NBUF)],
            [pltpu.SemaphoreType.DMA for _ in range(NBUF)],
        ],
    )
    def agg(src_hbm, dst_hbm, feat_hbm, zeros_hbm, out_hbm,
            src_v, dst_v, rows, acc, gsem, ssem):
        cid = lax.axis_index("c")
        sid = lax.axis_index("s")

        pltpu.sync_copy(zeros_hbm, acc.at[pl.ds(sid * ZROWS, ZROWS)])
        pltpu.sync_copy(src_hbm.at[cid, sid], src_v)
        pltpu.sync_copy(dst_hbm.at[sid], dst_v)
        plsc.subcore_barrier()

        n_iter = n_chunks // (NBUF * GRP)

        for k in range(NBUF):
            pltpu.async_copy(feat_hbm.at[src_v.at[k]], rows[k], gsem[k])

        def body(i, _):
            base = NBUF * i
            for k in range(NBUF):
                pltpu.make_async_copy(
                    feat_hbm.at[src_v.at[base + k]], rows[k], gsem[k]).wait()
                pltpu.async_copy(
                    rows[k], acc.at[dst_v.at[base + k]], ssem[k], add=True)
            for k in range(NBUF):
                pltpu.make_async_copy(
                    rows[k], acc.at[dst_v.at[base + k]], ssem[k]).wait()

                @pl.when(i + 1 < n_iter)
                def _():
                    pltpu.async_copy(
                        feat_hbm.at[src_v.at[base + NBUF + k]], rows[k],
                        gsem[k])
            return 0

        lax.fori_loop(0, n_iter, body, 0)
        plsc.subcore_barrier()

        pltpu.sync_copy(
            acc.at[pl.ds(sid * ZROWS, ZROWS)],
            out_hbm.at[cid, pl.ds(sid * ZROWS, ZROWS)],
        )

    return agg


def _dense_body(acc_ref, feat_ref, wn_ref, ws_ref, b_ref, out_ref):
    out_ref[...] = (
        jnp.dot(acc_ref[0], wn_ref[0:DH, :], preferred_element_type=jnp.float32)
        + jnp.dot(acc_ref[1], wn_ref[DH:D, :], preferred_element_type=jnp.float32)
        + jnp.dot(feat_ref[...], ws_ref[...], preferred_element_type=jnp.float32)
        + b_ref[...]
    )


def _make_dense(blk, n_blk):
    return pl.pallas_call(
        _dense_body,
        grid=(n_blk,),
        in_specs=[
            pl.BlockSpec((NC, blk, DH), lambda i: (0, i, 0)),
            pl.BlockSpec((blk, D), lambda i: (i, 0)),
            pl.BlockSpec((D, D), lambda i: (0, 0)),
            pl.BlockSpec((D, D), lambda i: (0, 0)),
            pl.BlockSpec((1, D), lambda i: (0, 0)),
        ],
        out_specs=pl.BlockSpec((blk, D), lambda i: (i, 0)),
        out_shape=jax.ShapeDtypeStruct((N_NODES, D), jnp.float32),
    )


def kernel(feat, edge_index, W_neigh, b_neigh, W_self):
    src = edge_index[0].astype(jnp.int32)
    dst = edge_index[1].astype(jnp.int32)
    n_edges = src.shape[0]

    per_tile = -(-n_edges // NS)
    n_chunks = -(-per_tile // (CHUNK * NBUF * GRP)) * (NBUF * GRP)
    e_pad = NS * n_chunks * CHUNK

    pad = e_pad - n_edges
    pad_idx = jnp.arange(pad, dtype=jnp.int32)
    src_p = jnp.concatenate([src, pad_idx % N_NODES])
    src_all = jnp.stack([2 * src_p, 2 * src_p + 1]).reshape(
        NC, NS, n_chunks // GRP, GRP * CHUNK)
    dst_p = jnp.concatenate(
        [dst, N_NODES + (pad_idx & 63)]
    ).reshape(NS, n_chunks // GRP, GRP * CHUNK)

    feat_half = feat.reshape(NC * N_NODES, DH)
    zeros = jnp.zeros((ZROWS, DH), jnp.float32)
    acc = _make_agg(n_chunks)(src_all, dst_p, feat_half, zeros)

    blk = 1000
    n_blk = N_NODES // blk
    return _make_dense(blk, n_blk)(acc, feat, W_neigh, W_self,
                                   b_neigh.reshape(1, D))
